# Initial kernel scaffold; baseline (speedup 1.0000x reference)
import jax
import jax.numpy as jnp
from jax.experimental import pallas as pl


def kernel(states, mask):
    raise NotImplementedError("write your pallas kernel here")


# speedup vs baseline: 51.6999x; 51.6999x over previous
import jax
import jax.numpy as jnp
from jax import lax
from jax.experimental import pallas as pl


def _first_valid_body(mask_ref, out_ref):
    m = mask_ref[...]
    nv = m.shape[1]
    col = lax.broadcasted_iota(jnp.int32, m.shape, 1)
    idx = jnp.where(m, col, nv)
    out_ref[...] = jnp.min(idx, axis=1)


def kernel(states, mask):
    b = states.shape[0]
    return pl.pallas_call(
        _first_valid_body,
        out_shape=jax.ShapeDtypeStruct((b,), jnp.int32),
    )(mask)
